# baseline (speedup 1.0000x reference)
import jax
import jax.numpy as jnp
from jax import lax
from jax.experimental import pallas as pl
from jax.experimental.pallas import tpu as pltpu
from jax.experimental.pallas import tpu_sc as plsc

_N = 10000
_D = 128
_E = 320000
_NP = 10240
_K = 128
_CH = 80
_CH0 = 160
_CH1 = 0
_CC0 = 80
_CC1 = 80
_NB = 8
_NC, _NS = 2, 16
_NW = _NC * _NS
_EPT = _CH * _K
_EP = _NW * _EPT
_RT = _NP // _NS
_BR = 1024
_G = _NP // _BR

_mesh = plsc.VectorSubcoreMesh(core_axis_name="c", subcore_axis_name="s")


def _fill(ref, val):
    v = jnp.full((16,), val, jnp.float32)
    for i in range(_K):
        for j in range(_D // 16):
            ref[i, pl.ds(j * 16, 16)] = v


def _sc_agg_body(y, srcr, dstr, out, acc, src_b, dst_b, rows0, rows1,
                 sem0, sem1):
    cid = lax.axis_index("c")
    sid = lax.axis_index("s")
    w = cid * _NS + sid

    _fill(rows0, 0.0)
    for k in range(_RT // _K):
        pltpu.sync_copy(rows0, acc.at[pl.ds(sid * _RT + k * _K, _K)])
    plsc.subcore_barrier()

    bufs = (rows0, rows1)
    sems = (sem0, sem1)
    row0 = lax.select(cid == 0, sid * _CH0, _NS * _CH0 + sid * _CH1)
    ng = lax.select(cid == 0, _CH0 // _NB, _CH1 // _NB)

    def _group(g, c):
        g0 = row0 + g * _NB
        pltpu.sync_copy(srcr.at[pl.ds(g0, _NB)], src_b)
        pltpu.sync_copy(dstr.at[pl.ds(g0, _NB)], dst_b)
        cp = pltpu.async_copy(y.at[src_b.at[0]], bufs[0], sems[0])
        for j in range(_NB):
            if j + 1 < _NB:
                cp_next = pltpu.async_copy(
                    y.at[src_b.at[j + 1]], bufs[(j + 1) % 2],
                    sems[(j + 1) % 2])
            cp.wait()
            pltpu.sync_copy(bufs[j % 2], acc.at[dst_b.at[j]], add=True)
            if j + 1 < _NB:
                cp = cp_next
        return c
    lax.fori_loop(0, ng, _group, 0)
    plsc.subcore_barrier()

    for k in range(_RT // _K):
        r0 = sid * _RT + k * _K
        pltpu.sync_copy(acc.at[pl.ds(r0, _K)], rows0)
        pltpu.sync_copy(rows0, out.at[pl.ds(cid * _NP + r0, _K)])


def _sc_cnt_body(dstr, out, acc, dst_b, rows_v, sem):
    cid = lax.axis_index("c")
    sid = lax.axis_index("s")
    w = cid * _NS + sid

    _fill(rows_v, 0.0)
    for k in range(_RT // _K):
        pltpu.sync_copy(rows_v, acc.at[pl.ds(sid * _RT + k * _K, _K)])
    plsc.subcore_barrier()

    _fill(rows_v, 1.0)
    row0 = lax.select(cid == 0, sid * _CC0, _NS * _CC0 + sid * _CC1)
    ng = lax.select(cid == 0, _CC0 // _NB, _CC1 // _NB)

    def _group(g, c):
        g0 = row0 + g * _NB
        pltpu.sync_copy(dstr.at[pl.ds(g0, _NB)], dst_b)
        for j in range(_NB):
            pltpu.sync_copy(rows_v, acc.at[dst_b.at[j]], add=True)
        return c
    lax.fori_loop(0, ng, _group, 0)
    plsc.subcore_barrier()

    for k in range(_RT // _K):
        r0 = sid * _RT + k * _K
        pltpu.sync_copy(acc.at[pl.ds(r0, _K)], rows_v)
        pltpu.sync_copy(rows_v, out.at[pl.ds(cid * _NP + r0, _K)])


_agg = pl.kernel(
    _sc_agg_body,
    out_type=[jax.ShapeDtypeStruct((_NC * _NP, _D), jnp.float32)],
    mesh=_mesh,
    scratch_types=[
        pltpu.VMEM_SHARED((_NP, _D), jnp.float32),
        pltpu.VMEM((_NB, _K), jnp.int32),
        pltpu.VMEM((_NB, _K), jnp.int32),
        pltpu.VMEM((_K, _D), jnp.float32),
        pltpu.VMEM((_K, _D), jnp.float32),
        pltpu.SemaphoreType.DMA,
        pltpu.SemaphoreType.DMA,
    ],
)

_cnt_kernel = pl.kernel(
    _sc_cnt_body,
    out_type=[jax.ShapeDtypeStruct((_NC * _NP, _D), jnp.float32)],
    mesh=_mesh,
    scratch_types=[
        pltpu.VMEM_SHARED((_NP, _D), jnp.float32),
        pltpu.VMEM((_NB, _K), jnp.int32),
        pltpu.VMEM((_K, _D), jnp.float32),
        pltpu.SemaphoreType.DMA,
    ],
)


_DN = (((1,), (1,)), ((), ()))


def _mm2_body(x_ref, wl_ref, bl_ref, wr_ref, y_ref, z_ref):
    x = x_ref[...]
    y_ref[...] = lax.dot_general(x, wl_ref[...], _DN,
                                 preferred_element_type=jnp.float32)
    z_ref[...] = lax.dot_general(x, wr_ref[...], _DN,
                                 preferred_element_type=jnp.float32) + bl_ref[...]


def _mid1_body(p0_ref, p1_ref, c0_ref, c1_ref, z_ref, wl_ref, bl_ref, wr_ref,
               y_ref, zo_ref, inv_ref):
    inv = 1.0 / jnp.maximum(c0_ref[...] + c1_ref[...], 1.0)
    h = jnp.maximum((p0_ref[...] + p1_ref[...]) * inv + z_ref[...], 0.0)
    inv_ref[...] = inv
    y_ref[...] = lax.dot_general(h, wl_ref[...], _DN,
                                 preferred_element_type=jnp.float32)
    zo_ref[...] = lax.dot_general(h, wr_ref[...], _DN,
                                  preferred_element_type=jnp.float32) + bl_ref[...]


def _mid2_body(p0_ref, p1_ref, inv_ref, z_ref, wl_ref, bl_ref, wr_ref,
               y_ref, zo_ref):
    h = (p0_ref[...] + p1_ref[...]) * inv_ref[...] + z_ref[...]
    y_ref[...] = lax.dot_general(h, wl_ref[...], _DN,
                                 preferred_element_type=jnp.float32)
    zo_ref[...] = lax.dot_general(h, wr_ref[...], _DN,
                                  preferred_element_type=jnp.float32) + bl_ref[...]


def _fin_body(p0_ref, p1_ref, inv_ref, z_ref, o_ref):
    o_ref[...] = (p0_ref[...] + p1_ref[...]) * inv_ref[...] + z_ref[...]


_row_spec = pl.BlockSpec((_BR, _D), lambda i: (i, 0))
_p0_spec = pl.BlockSpec((_BR, _D), lambda i: (i, 0))
_p1_spec = pl.BlockSpec((_BR, _D), lambda i: (_G + i, 0))
_w_spec = pl.BlockSpec((_D, _D), lambda i: (0, 0))
_b_spec = pl.BlockSpec((1, _D), lambda i: (0, 0))
_nd_f32 = jax.ShapeDtypeStruct((_NP, _D), jnp.float32)

_mm2 = pl.pallas_call(
    _mm2_body,
    grid=(_G,),
    in_specs=[_row_spec, _w_spec, _b_spec, _w_spec],
    out_specs=[_row_spec, _row_spec],
    out_shape=[_nd_f32, _nd_f32],
)

_mid1 = pl.pallas_call(
    _mid1_body,
    grid=(_G,),
    in_specs=[_p0_spec, _p1_spec, _p0_spec, _p1_spec, _row_spec,
              _w_spec, _b_spec, _w_spec],
    out_specs=[_row_spec, _row_spec, _row_spec],
    out_shape=[_nd_f32, _nd_f32, _nd_f32],
)

_mid2 = pl.pallas_call(
    _mid2_body,
    grid=(_G,),
    in_specs=[_p0_spec, _p1_spec, _row_spec, _row_spec, _w_spec, _b_spec,
              _w_spec],
    out_specs=[_row_spec, _row_spec],
    out_shape=[_nd_f32, _nd_f32],
)

_fin = pl.pallas_call(
    _fin_body,
    grid=(_G,),
    in_specs=[_p0_spec, _p1_spec, _row_spec, _row_spec],
    out_specs=[_row_spec],
    out_shape=[_nd_f32],
)


def kernel(x, edge_index, Wl1, bl1, Wr1, Wl2, bl2, Wr2, Wl3, bl3, Wr3):
    src = edge_index[0]
    dst = edge_index[1]
    pad_e = _EP - _E
    srcp = jnp.concatenate([src, jnp.zeros((pad_e,), jnp.int32)])
    srcp = srcp.reshape(_EP // _K, _K)
    dstp = jnp.concatenate([dst, jnp.full((pad_e,), _N, jnp.int32)])
    dstp = dstp.reshape(_EP // _K, _K)
    xp = jnp.concatenate([x, jnp.zeros((_NP - _N, _D), jnp.float32)])
    bl1r = bl1.reshape(1, _D)
    bl2r = bl2.reshape(1, _D)
    bl3r = bl3.reshape(1, _D)

    (cnt,) = _cnt_kernel(dstp)
    y1, z1 = _mm2(xp, Wl1, bl1r, Wr1)
    (p1,) = _agg(y1, srcp, dstp)
    y2, z2, inv = _mid1(p1, p1, cnt, cnt, z1, Wl2, bl2r, Wr2)
    (p2,) = _agg(y2, srcp, dstp)
    y3, z3 = _mid2(p2, p2, inv, z2, Wl3, bl3r, Wr3)
    (p3,) = _agg(y3, srcp, dstp)
    (out,) = _fin(p3, p3, inv, z3)
    return out[:_N]


# speedup vs baseline: 1.2817x; 1.2817x over previous
import jax
import jax.numpy as jnp
from jax import lax
from jax.experimental import pallas as pl
from jax.experimental.pallas import tpu as pltpu
from jax.experimental.pallas import tpu_sc as plsc

_N = 10000
_D = 128
_E = 320000
_NP = 10240
_K = 128
_CH = 80
_CH0 = 120
_CH1 = 40
_NB = 8
_NC, _NS = 2, 16
_NW = _NC * _NS
_EPT = _CH * _K
_EP = _NW * _EPT
_RT = _NP // _NS
_BR = 1024
_G = _NP // _BR

_mesh = plsc.VectorSubcoreMesh(core_axis_name="c", subcore_axis_name="s")


def _fill(ref, val):
    v = jnp.full((16,), val, jnp.float32)
    rows, cols = ref.shape
    for i in range(rows):
        for j in range(cols // 16):
            ref[i, pl.ds(j * 16, 16)] = v


def _sc_agg_body(y, srcr, dstr, out, acc, src_b, dst_b, rows0, rows1,
                 sem0, sem1):
    cid = lax.axis_index("c")
    sid = lax.axis_index("s")

    _fill(rows0, 0.0)
    for k in range(_RT // _K):
        pltpu.sync_copy(rows0, acc.at[pl.ds(sid * _RT + k * _K, _K)])
    plsc.subcore_barrier()

    bufs = (rows0, rows1)
    sems = (sem0, sem1)
    row0 = lax.select(cid == 0, sid * _CH0, _NS * _CH0 + sid * _CH1)
    ng = lax.select(cid == 0, _CH0 // _NB, _CH1 // _NB)

    def _group(g, c):
        g0 = row0 + g * _NB
        pltpu.sync_copy(srcr.at[pl.ds(g0, _NB)], src_b)
        pltpu.sync_copy(dstr.at[pl.ds(g0, _NB)], dst_b)
        cp = pltpu.async_copy(y.at[src_b.at[0]], bufs[0], sems[0])
        for j in range(_NB):
            if j + 1 < _NB:
                cp_next = pltpu.async_copy(
                    y.at[src_b.at[j + 1]], bufs[(j + 1) % 2],
                    sems[(j + 1) % 2])
            cp.wait()
            pltpu.sync_copy(bufs[j % 2], acc.at[dst_b.at[j]], add=True)
            if j + 1 < _NB:
                cp = cp_next
        return c
    lax.fori_loop(0, ng, _group, 0)
    plsc.subcore_barrier()

    for k in range(_RT // _K):
        r0 = sid * _RT + k * _K
        pltpu.sync_copy(acc.at[pl.ds(r0, _K)], rows0)
        pltpu.sync_copy(rows0, out.at[pl.ds(cid * _NP + r0, _K)])


def _sc_cnt_body(dstr, out, acc, dst_b, rows_v, sem):
    cid = lax.axis_index("c")
    sid = lax.axis_index("s")

    _fill(rows_v, 0.0)
    for k in range(_RT // _K):
        pltpu.sync_copy(rows_v, acc.at[pl.ds(sid * _RT + k * _K, _K)])
    plsc.subcore_barrier()

    _fill(rows_v, 1.0)
    row0 = cid * _NS * _CH + sid * _CH
    ng = _CH // _NB

    def _group(g, c):
        g0 = row0 + g * _NB
        pltpu.sync_copy(dstr.at[pl.ds(g0, _NB)], dst_b)
        for j in range(_NB):
            pltpu.sync_copy(rows_v, acc.at[dst_b.at[j]], add=True)
        return c
    lax.fori_loop(0, ng, _group, 0)
    plsc.subcore_barrier()

    for k in range(_RT // _K):
        r0 = sid * _RT + k * _K
        pltpu.sync_copy(acc.at[pl.ds(r0, _K)], rows_v)
        pltpu.sync_copy(rows_v, out.at[pl.ds(cid * _NP + r0, _K)])


_agg = pl.kernel(
    _sc_agg_body,
    out_type=[jax.ShapeDtypeStruct((_NC * _NP, _D), jnp.float32)],
    mesh=_mesh,
    scratch_types=[
        pltpu.VMEM_SHARED((_NP, _D), jnp.float32),
        pltpu.VMEM((_NB, _K), jnp.int32),
        pltpu.VMEM((_NB, _K), jnp.int32),
        pltpu.VMEM((_K, _D), jnp.float32),
        pltpu.VMEM((_K, _D), jnp.float32),
        pltpu.SemaphoreType.DMA,
        pltpu.SemaphoreType.DMA,
    ],
)

_cnt_kernel = pl.kernel(
    _sc_cnt_body,
    out_type=[jax.ShapeDtypeStruct((_NC * _NP, _D), jnp.float32)],
    mesh=_mesh,
    scratch_types=[
        pltpu.VMEM_SHARED((_NP, _D), jnp.float32),
        pltpu.VMEM((_NB, _K), jnp.int32),
        pltpu.VMEM((_K, _D), jnp.float32),
        pltpu.SemaphoreType.DMA,
    ],
)


_DN = (((1,), (1,)), ((), ()))


def _mm2_body(x_ref, wl_ref, bl_ref, wr_ref, y_ref, z_ref):
    x = x_ref[...]
    y_ref[...] = lax.dot_general(x, wl_ref[...], _DN,
                                 preferred_element_type=jnp.float32)
    z_ref[...] = lax.dot_general(x, wr_ref[...], _DN,
                                 preferred_element_type=jnp.float32) + bl_ref[...]


def _mid1_body(p0_ref, p1_ref, c0_ref, c1_ref, z_ref, wl_ref, bl_ref, wr_ref,
               y_ref, zo_ref, inv_ref):
    inv = 1.0 / jnp.maximum(c0_ref[...] + c1_ref[...], 1.0)
    h = jnp.maximum((p0_ref[...] + p1_ref[...]) * inv + z_ref[...], 0.0)
    inv_ref[...] = inv
    y_ref[...] = lax.dot_general(h, wl_ref[...], _DN,
                                 preferred_element_type=jnp.float32)
    zo_ref[...] = lax.dot_general(h, wr_ref[...], _DN,
                                  preferred_element_type=jnp.float32) + bl_ref[...]


def _mid2_body(p0_ref, p1_ref, inv_ref, z_ref, wl_ref, bl_ref, wr_ref,
               y_ref, zo_ref):
    h = (p0_ref[...] + p1_ref[...]) * inv_ref[...] + z_ref[...]
    y_ref[...] = lax.dot_general(h, wl_ref[...], _DN,
                                 preferred_element_type=jnp.float32)
    zo_ref[...] = lax.dot_general(h, wr_ref[...], _DN,
                                  preferred_element_type=jnp.float32) + bl_ref[...]


def _fin_body(p0_ref, p1_ref, inv_ref, z_ref, o_ref):
    o_ref[...] = (p0_ref[...] + p1_ref[...]) * inv_ref[...] + z_ref[...]


_row_spec = pl.BlockSpec((_BR, _D), lambda i: (i, 0))
_p0_spec = pl.BlockSpec((_BR, _D), lambda i: (i, 0))
_p1_spec = pl.BlockSpec((_BR, _D), lambda i: (_G + i, 0))
_w_spec = pl.BlockSpec((_D, _D), lambda i: (0, 0))
_b_spec = pl.BlockSpec((1, _D), lambda i: (0, 0))
_nd_f32 = jax.ShapeDtypeStruct((_NP, _D), jnp.float32)

_mm2 = pl.pallas_call(
    _mm2_body,
    grid=(_G,),
    in_specs=[_row_spec, _w_spec, _b_spec, _w_spec],
    out_specs=[_row_spec, _row_spec],
    out_shape=[_nd_f32, _nd_f32],
)

_mid1 = pl.pallas_call(
    _mid1_body,
    grid=(_G,),
    in_specs=[_p0_spec, _p1_spec, _p0_spec, _p1_spec, _row_spec,
              _w_spec, _b_spec, _w_spec],
    out_specs=[_row_spec, _row_spec, _row_spec],
    out_shape=[_nd_f32, _nd_f32, _nd_f32],
)

_mid2 = pl.pallas_call(
    _mid2_body,
    grid=(_G,),
    in_specs=[_p0_spec, _p1_spec, _row_spec, _row_spec, _w_spec, _b_spec,
              _w_spec],
    out_specs=[_row_spec, _row_spec],
    out_shape=[_nd_f32, _nd_f32],
)

_fin = pl.pallas_call(
    _fin_body,
    grid=(_G,),
    in_specs=[_p0_spec, _p1_spec, _row_spec, _row_spec],
    out_specs=[_row_spec],
    out_shape=[_nd_f32],
)


def kernel(x, edge_index, Wl1, bl1, Wr1, Wl2, bl2, Wr2, Wl3, bl3, Wr3):
    src = edge_index[0]
    dst = edge_index[1]
    pad_e = _EP - _E
    srcp = jnp.concatenate([src, jnp.zeros((pad_e,), jnp.int32)])
    srcp = srcp.reshape(_EP // _K, _K)
    dstp = jnp.concatenate([dst, jnp.full((pad_e,), _N, jnp.int32)])
    dstp = dstp.reshape(_EP // _K, _K)
    xp = jnp.concatenate([x, jnp.zeros((_NP - _N, _D), jnp.float32)])
    bl1r = bl1.reshape(1, _D)
    bl2r = bl2.reshape(1, _D)
    bl3r = bl3.reshape(1, _D)

    (cnt,) = _cnt_kernel(dstp)
    y1, z1 = _mm2(xp, Wl1, bl1r, Wr1)
    (p1,) = _agg(y1, srcp, dstp)
    y2, z2, inv = _mid1(p1, p1, cnt, cnt, z1, Wl2, bl2r, Wr2)
    (p2,) = _agg(y2, srcp, dstp)
    y3, z3 = _mid2(p2, p2, inv, z2, Wl3, bl3r, Wr3)
    (p3,) = _agg(y3, srcp, dstp)
    (out,) = _fin(p3, p3, inv, z3)
    return out[:_N]
